# Initial kernel scaffold; baseline (speedup 1.0000x reference)
import jax
import jax.numpy as jnp
from jax.experimental import pallas as pl


def kernel(edge_index, features, preference, W1, b1, W2, b2, Wc, bc):
    raise NotImplementedError("write your pallas kernel here")


# speedup vs baseline: 9.4028x; 9.4028x over previous
import functools

import jax
import jax.numpy as jnp
from jax import lax
from jax.experimental import pallas as pl
from jax.experimental.pallas import tpu as pltpu
from jax.experimental.pallas import tpu_sc as plsc

N_USER = 10000
N_ITEM = 40000
N_NODES = N_USER + N_ITEM
DIM = 64
FEAT = 128
HID = 256
N_EDGES = 800000

NC = 2
NS = 16
HALF = N_NODES // NC
ACC_ROWS = HALF + 8
CHUNK = 128
N_CHUNKS = N_EDGES // CHUNK
CHUNKS_PER_TILE = -(-N_CHUNKS // NS)
DEG_W = 8

_MESH = plsc.VectorSubcoreMesh(core_axis_name="c", subcore_axis_name="s")
_SC_PARAMS = pltpu.CompilerParams(use_tc_tiling_on_sc=False)


def _deg_body(col_hbm, ones_hbm, zeros_hbm, out_hbm,
              colbuf, idxbuf, ones_v, acc):
    c = lax.axis_index("c")
    s = lax.axis_index("s")
    base = c * HALF

    @pl.when(s == 0)
    def _init():
        pltpu.sync_copy(zeros_hbm, acc)
    pltpu.sync_copy(ones_hbm, ones_v)
    plsc.subcore_barrier()

    def step(i, carry):
        b = s + i * NS

        @pl.when(b < N_CHUNKS)
        def _():
            pltpu.sync_copy(col_hbm.at[pl.ds(b * CHUNK, CHUNK)], colbuf)
            for j in range(CHUNK // 16):
                v = colbuf[pl.ds(j * 16, 16)]
                t = v - base
                ok = (t >= 0) & (t < HALF)
                idxbuf[pl.ds(j * 16, 16)] = jnp.where(ok, t, HALF)
            pltpu.sync_copy(ones_v, acc.at[idxbuf], add=True)
        return carry

    lax.fori_loop(0, CHUNKS_PER_TILE, step, 0)
    plsc.subcore_barrier()

    @pl.when(s == 0)
    def _flush():
        pltpu.sync_copy(acc, out_hbm.at[c])


def _degree_histogram(col):
    ones = jnp.ones((CHUNK, DEG_W), jnp.float32)
    zeros = jnp.zeros((ACC_ROWS, DEG_W), jnp.float32)
    fn = pl.kernel(
        _deg_body,
        out_type=jax.ShapeDtypeStruct((NC, ACC_ROWS, DEG_W), jnp.float32),
        mesh=_MESH,
        scratch_types=[
            pltpu.VMEM((CHUNK,), jnp.int32),
            pltpu.VMEM((CHUNK,), jnp.int32),
            pltpu.VMEM((CHUNK, DEG_W), jnp.float32),
            pltpu.VMEM_SHARED((ACC_ROWS, DEG_W), jnp.float32),
        ],
        compiler_params=_SC_PARAMS,
    )
    return fn(col, ones, zeros)


def _conv_body(row_hbm, col_hbm, y_hbm, zeros_hbm, out_hbm,
               rowbuf, colbuf, idxbuf, rows_v, acc, sem):
    c = lax.axis_index("c")
    s = lax.axis_index("s")
    base = c * HALF

    @pl.when(s == 0)
    def _init():
        pltpu.sync_copy(zeros_hbm, acc)
    plsc.subcore_barrier()

    def step(i, carry):
        b = s + i * NS

        @pl.when(b < N_CHUNKS)
        def _():
            pltpu.sync_copy(row_hbm.at[pl.ds(b * CHUNK, CHUNK)], rowbuf)
            pltpu.sync_copy(col_hbm.at[pl.ds(b * CHUNK, CHUNK)], colbuf)
            for j in range(CHUNK // 16):
                v = colbuf[pl.ds(j * 16, 16)]
                t = v - base
                ok = (t >= 0) & (t < HALF)
                idxbuf[pl.ds(j * 16, 16)] = jnp.where(ok, t, HALF)
            pltpu.async_copy(y_hbm.at[rowbuf], rows_v, sem).wait()
            pltpu.sync_copy(rows_v, acc.at[idxbuf], add=True)
        return carry

    lax.fori_loop(0, CHUNKS_PER_TILE, step, 0)
    plsc.subcore_barrier()

    @pl.when(s == 0)
    def _flush():
        pltpu.sync_copy(acc, out_hbm.at[c])


def _edge_aggregate(row, col, y):
    zeros = jnp.zeros((ACC_ROWS, DIM), jnp.float32)
    fn = pl.kernel(
        _conv_body,
        out_type=jax.ShapeDtypeStruct((NC, ACC_ROWS, DIM), jnp.float32),
        mesh=_MESH,
        scratch_types=[
            pltpu.VMEM((CHUNK,), jnp.int32),
            pltpu.VMEM((CHUNK,), jnp.int32),
            pltpu.VMEM((CHUNK,), jnp.int32),
            pltpu.VMEM((CHUNK, DIM), jnp.float32),
            pltpu.VMEM_SHARED((ACC_ROWS, DIM), jnp.float32),
            pltpu.SemaphoreType.DMA,
        ],
        compiler_params=_SC_PARAMS,
    )
    out = fn(row, col, y, zeros)
    return out[:, :HALF, :].reshape(N_NODES, DIM)


def _mlp_norm_body(f_ref, w1_ref, b1_ref, w2_ref, b2_ref, o_ref):
    z = jnp.dot(f_ref[...], w1_ref[...], preferred_element_type=jnp.float32)
    z = z + b1_ref[...]
    z = jnp.where(z > 0, z, 0.01 * z)
    t = jnp.dot(z, w2_ref[...], preferred_element_type=jnp.float32) + b2_ref[...]
    n = jnp.sqrt(jnp.sum(t * t, axis=1, keepdims=True))
    o_ref[...] = t / jnp.maximum(n, 1e-12)


def _mlp_norm(features, W1, b1, W2, b2):
    blk, grid = 800, N_ITEM // 800
    return pl.pallas_call(
        _mlp_norm_body,
        grid=(grid,),
        in_specs=[
            pl.BlockSpec((blk, FEAT), lambda i: (i, 0)),
            pl.BlockSpec((FEAT, HID), lambda i: (0, 0)),
            pl.BlockSpec((1, HID), lambda i: (0, 0)),
            pl.BlockSpec((HID, DIM), lambda i: (0, 0)),
            pl.BlockSpec((1, DIM), lambda i: (0, 0)),
        ],
        out_specs=pl.BlockSpec((blk, DIM), lambda i: (i, 0)),
        out_shape=jax.ShapeDtypeStruct((N_ITEM, DIM), jnp.float32),
    )(features, W1, b1.reshape(1, HID), W2, b2.reshape(1, DIM))


def _norm_body(p_ref, o_ref):
    t = p_ref[...]
    n = jnp.sqrt(jnp.sum(t * t, axis=1, keepdims=True))
    o_ref[...] = t / jnp.maximum(n, 1e-12)


def _norm_rows(p):
    blk, grid = 1000, N_USER // 1000
    return pl.pallas_call(
        _norm_body,
        grid=(grid,),
        in_specs=[pl.BlockSpec((blk, DIM), lambda i: (i, 0))],
        out_specs=pl.BlockSpec((blk, DIM), lambda i: (i, 0)),
        out_shape=jax.ShapeDtypeStruct((N_USER, DIM), jnp.float32),
    )(p)


def _dinv(deg8_blk):
    d = deg8_blk[:, 0:1] + 1.0
    return lax.rsqrt(d)


def _scale_body(x_ref, deg_ref, wc_ref, y_ref, self_ref):
    xw = jnp.dot(x_ref[...], wc_ref[...], preferred_element_type=jnp.float32)
    di = _dinv(deg_ref[...])
    y_ref[...] = xw * di
    self_ref[...] = xw * (di * di)


def _scale(x, deg8, Wc):
    blk, grid = 1000, N_NODES // 1000
    return pl.pallas_call(
        _scale_body,
        grid=(grid,),
        in_specs=[
            pl.BlockSpec((blk, DIM), lambda i: (i, 0)),
            pl.BlockSpec((blk, DEG_W), lambda i: (i, 0)),
            pl.BlockSpec((DIM, DIM), lambda i: (0, 0)),
        ],
        out_specs=[
            pl.BlockSpec((blk, DIM), lambda i: (i, 0)),
            pl.BlockSpec((blk, DIM), lambda i: (i, 0)),
        ],
        out_shape=[
            jax.ShapeDtypeStruct((N_NODES, DIM), jnp.float32),
            jax.ShapeDtypeStruct((N_NODES, DIM), jnp.float32),
        ],
    )(x, deg8, Wc)


def _mid_body(s_ref, self_ref, deg_ref, wc_ref, bc_ref,
              h_ref, y_ref, self1_ref):
    di = _dinv(deg_ref[...])
    h = s_ref[...] * di + self_ref[...] + bc_ref[...]
    h_ref[...] = h
    hw = jnp.dot(h, wc_ref[...], preferred_element_type=jnp.float32)
    y_ref[...] = hw * di
    self1_ref[...] = hw * (di * di)


def _mid(s1, self0, deg8, Wc, bc):
    blk, grid = 1000, N_NODES // 1000
    return pl.pallas_call(
        _mid_body,
        grid=(grid,),
        in_specs=[
            pl.BlockSpec((blk, DIM), lambda i: (i, 0)),
            pl.BlockSpec((blk, DIM), lambda i: (i, 0)),
            pl.BlockSpec((blk, DEG_W), lambda i: (i, 0)),
            pl.BlockSpec((DIM, DIM), lambda i: (0, 0)),
            pl.BlockSpec((1, DIM), lambda i: (0, 0)),
        ],
        out_specs=[pl.BlockSpec((blk, DIM), lambda i: (i, 0))] * 3,
        out_shape=[jax.ShapeDtypeStruct((N_NODES, DIM), jnp.float32)] * 3,
    )(s1, self0, deg8, Wc, bc.reshape(1, DIM))


def _final_body(s_ref, self_ref, deg_ref, x_ref, h_ref, bc_ref, o_ref):
    di = _dinv(deg_ref[...])
    h1 = s_ref[...] * di + self_ref[...] + bc_ref[...]
    o_ref[...] = x_ref[...] + h_ref[...] + h1


def _final(s2, self1, deg8, x, h, bc):
    blk, grid = 1000, N_NODES // 1000
    return pl.pallas_call(
        _final_body,
        grid=(grid,),
        in_specs=[
            pl.BlockSpec((blk, DIM), lambda i: (i, 0)),
            pl.BlockSpec((blk, DIM), lambda i: (i, 0)),
            pl.BlockSpec((blk, DEG_W), lambda i: (i, 0)),
            pl.BlockSpec((blk, DIM), lambda i: (i, 0)),
            pl.BlockSpec((blk, DIM), lambda i: (i, 0)),
            pl.BlockSpec((1, DIM), lambda i: (0, 0)),
        ],
        out_specs=pl.BlockSpec((blk, DIM), lambda i: (i, 0)),
        out_shape=jax.ShapeDtypeStruct((N_NODES, DIM), jnp.float32),
    )(s2, self1, deg8, x, h, bc.reshape(1, DIM))


def kernel(edge_index, features, preference, W1, b1, W2, b2, Wc, bc):
    row = edge_index[0]
    col = edge_index[1]

    hist = _degree_histogram(col)
    deg8 = hist[:, :HALF, :].reshape(N_NODES, DEG_W)

    x_items = _mlp_norm(features, W1, b1, W2, b2)
    x_pref = _norm_rows(preference)
    x = jnp.concatenate([x_pref, x_items], axis=0)

    y0, self0 = _scale(x, deg8, Wc)
    s1 = _edge_aggregate(row, col, y0)
    h, y1, self1 = _mid(s1, self0, deg8, Wc, bc)
    s2 = _edge_aggregate(row, col, y1)
    x_hat = _final(s2, self1, deg8, x, h, bc)
    return (x_hat, preference)


# speedup vs baseline: 15.1600x; 1.6123x over previous
import jax
import jax.numpy as jnp
from jax import lax
from jax.experimental import pallas as pl
from jax.experimental.pallas import tpu as pltpu
from jax.experimental.pallas import tpu_sc as plsc

N_USER = 10000
N_ITEM = 40000
N_NODES = N_USER + N_ITEM
DIM = 64
HDIM = 32
FEAT = 128
HID = 256
N_EDGES = 800000

NC = 2
NS = 16
NT = NC * NS
CHUNK = 128
EPT = 25088
PAD_E = NT * EPT
NCH = EPT // CHUNK
ACC_ROWS = N_NODES + 8
K = 2
SLOTS = 2 * K
HSLOTS = 4

_PP = N_NODES << 16
PAD_PACKED = _PP - (1 << 32) if _PP >= (1 << 31) else _PP

_MESH = plsc.VectorSubcoreMesh(core_axis_name="c", subcore_axis_name="s")
_SC_PARAMS = pltpu.CompilerParams(use_tc_tiling_on_sc=False)


def _hist_body(pk_hbm, ones_hbm, zeros_hbm, out_hbm,
               ebuf, idxsl, ones_v, acc, sem_s, sem_z):
    c = lax.axis_index("c")
    s = lax.axis_index("s")

    @pl.when(s == 0)
    def _init():
        pltpu.async_copy(zeros_hbm, acc, sem_z)
    pltpu.sync_copy(ones_hbm, ones_v)
    g = c * NS + s
    pltpu.sync_copy(pk_hbm.at[pl.ds(g * EPT, EPT)], ebuf)

    @pl.when(s == 0)
    def _initw():
        pltpu.make_async_copy(zeros_hbm, acc, sem_z).wait()
    plsc.subcore_barrier()

    def step(i, carry):
        sl = lax.rem(i, HSLOTS)

        @pl.when(i >= HSLOTS)
        def _reclaim():
            pltpu.make_async_copy(
                ones_v, acc.at[idxsl.at[sl]], sem_s.at[sl]).wait()
        for j in range(CHUNK // 16):
            p = ebuf[pl.ds(i * CHUNK + j * 16, 16)]
            idxsl[sl, pl.ds(j * 16, 16)] = lax.shift_right_logical(p, 16)
        pltpu.async_copy(ones_v, acc.at[idxsl.at[sl]], sem_s.at[sl], add=True)
        return carry

    lax.fori_loop(0, NCH, step, 0)

    def drain(i, carry):
        sl = lax.rem(i, HSLOTS)
        pltpu.make_async_copy(
            ones_v, acc.at[idxsl.at[sl]], sem_s.at[sl]).wait()
        return carry

    lax.fori_loop(NCH - HSLOTS, NCH, drain, 0)
    plsc.subcore_barrier()

    @pl.when(s == 0)
    def _flush():
        pltpu.sync_copy(acc, out_hbm.at[c])


def _degree_histogram(packed):
    ones = jnp.ones((CHUNK, 8), jnp.float32)
    zeros = jnp.zeros((ACC_ROWS, 8), jnp.float32)
    fn = pl.kernel(
        _hist_body,
        out_type=jax.ShapeDtypeStruct((NC, ACC_ROWS, 8), jnp.float32),
        mesh=_MESH,
        scratch_types=[
            pltpu.VMEM((EPT,), jnp.int32),
            pltpu.VMEM((HSLOTS, CHUNK), jnp.int32),
            pltpu.VMEM((CHUNK, 8), jnp.float32),
            pltpu.VMEM_SHARED((ACC_ROWS, 8), jnp.float32),
            pltpu.SemaphoreType.DMA((HSLOTS,)),
            pltpu.SemaphoreType.DMA,
        ],
        compiler_params=_SC_PARAMS,
    )
    return fn(packed, ones, zeros)


def _conv_body(pk_hbm, y_hbm, zeros_hbm, out_hbm,
               pkbuf, idxsl, rowsl, rows_v, acc, sem_p, sem_g, sem_s, sem_z):
    c = lax.axis_index("c")
    s = lax.axis_index("s")

    @pl.when(s == 0)
    def _init():
        pltpu.async_copy(zeros_hbm, acc, sem_z)
    ebase = s * (2 * EPT)
    rowbase = c * N_NODES

    def fire_pk(t):
        sl = lax.rem(t, SLOTS)
        pltpu.async_copy(pk_hbm.at[pl.ds(ebase + t * CHUNK, CHUNK)],
                         pkbuf.at[sl], sem_p.at[sl])

    def unpack_fire(t):
        sl = lax.rem(t, SLOTS)
        pltpu.make_async_copy(
            pk_hbm.at[pl.ds(ebase + t * CHUNK, CHUNK)],
            pkbuf.at[sl], sem_p.at[sl]).wait()
        for j in range(CHUNK // 16):
            p = pkbuf[sl, pl.ds(j * 16, 16)]
            idxsl[sl, pl.ds(j * 16, 16)] = lax.shift_right_logical(p, 16)
            rowsl[sl, pl.ds(j * 16, 16)] = (p & 0xFFFF) + rowbase
        pltpu.async_copy(y_hbm.at[rowsl.at[sl]], rows_v.at[sl], sem_g.at[sl])

    for b in range(SLOTS):
        fire_pk(jnp.int32(b))

    @pl.when(s == 0)
    def _initw():
        pltpu.make_async_copy(zeros_hbm, acc, sem_z).wait()
    plsc.subcore_barrier()

    def step(i, carry):
        sl = lax.rem(i, SLOTS)
        unpack_fire(i)
        pltpu.make_async_copy(
            y_hbm.at[rowsl.at[sl]], rows_v.at[sl], sem_g.at[sl]).wait()
        pltpu.async_copy(
            rows_v.at[sl], acc.at[idxsl.at[sl]], sem_s.at[sl], add=True)
        pltpu.make_async_copy(
            rows_v.at[sl], acc.at[idxsl.at[sl]], sem_s.at[sl]).wait()
        t3 = i + SLOTS

        @pl.when(t3 < 2 * NCH)
        def _refill():
            fire_pk(t3)
        return carry

    lax.fori_loop(0, 2 * NCH, step, 0)
    plsc.subcore_barrier()

    @pl.when(s == 0)
    def _flush():
        pltpu.sync_copy(acc, out_hbm.at[c])


def _edge_aggregate(packed, y):
    zeros = jnp.zeros((ACC_ROWS, HDIM), jnp.float32)
    fn = pl.kernel(
        _conv_body,
        out_type=jax.ShapeDtypeStruct((NC, ACC_ROWS, HDIM), jnp.float32),
        mesh=_MESH,
        scratch_types=[
            pltpu.VMEM((SLOTS, CHUNK), jnp.int32),
            pltpu.VMEM((SLOTS, CHUNK), jnp.int32),
            pltpu.VMEM((SLOTS, CHUNK), jnp.int32),
            pltpu.VMEM((SLOTS, CHUNK, HDIM), jnp.float32),
            pltpu.VMEM_SHARED((ACC_ROWS, HDIM), jnp.float32),
            pltpu.SemaphoreType.DMA((SLOTS,)),
            pltpu.SemaphoreType.DMA((SLOTS,)),
            pltpu.SemaphoreType.DMA((SLOTS,)),
            pltpu.SemaphoreType.DMA,
        ],
        compiler_params=_SC_PARAMS,
    )
    return fn(packed, y, zeros)


def _pack_body(r_ref, c_ref, o_ref):
    o_ref[...] = (c_ref[...] << 16) | r_ref[...]


def _pack_edges(row, col):
    out = pl.pallas_call(
        _pack_body,
        out_shape=jax.ShapeDtypeStruct((1000, 800), jnp.int32),
    )(row.reshape(1000, 800), col.reshape(1000, 800))
    return out.reshape(N_EDGES)


def _mlp_norm_body(f_ref, w1_ref, b1_ref, w2_ref, b2_ref, o_ref):
    z = jnp.dot(f_ref[...], w1_ref[...], preferred_element_type=jnp.float32)
    z = z + b1_ref[...]
    z = jnp.where(z > 0, z, 0.01 * z)
    t = jnp.dot(z, w2_ref[...], preferred_element_type=jnp.float32) + b2_ref[...]
    n = jnp.sqrt(jnp.sum(t * t, axis=1, keepdims=True))
    o_ref[...] = t / jnp.maximum(n, 1e-12)


def _mlp_norm(features, W1, b1, W2, b2):
    blk, grid = 800, N_ITEM // 800
    return pl.pallas_call(
        _mlp_norm_body,
        grid=(grid,),
        in_specs=[
            pl.BlockSpec((blk, FEAT), lambda i: (i, 0)),
            pl.BlockSpec((FEAT, HID), lambda i: (0, 0)),
            pl.BlockSpec((1, HID), lambda i: (0, 0)),
            pl.BlockSpec((HID, DIM), lambda i: (0, 0)),
            pl.BlockSpec((1, DIM), lambda i: (0, 0)),
        ],
        out_specs=pl.BlockSpec((blk, DIM), lambda i: (i, 0)),
        out_shape=jax.ShapeDtypeStruct((N_ITEM, DIM), jnp.float32),
    )(features, W1, b1.reshape(1, HID), W2, b2.reshape(1, DIM))


def _norm_body(p_ref, o_ref):
    t = p_ref[...]
    n = jnp.sqrt(jnp.sum(t * t, axis=1, keepdims=True))
    o_ref[...] = t / jnp.maximum(n, 1e-12)


def _norm_rows(p):
    blk, grid = 1000, N_USER // 1000
    return pl.pallas_call(
        _norm_body,
        grid=(grid,),
        in_specs=[pl.BlockSpec((blk, DIM), lambda i: (i, 0))],
        out_specs=pl.BlockSpec((blk, DIM), lambda i: (i, 0)),
        out_shape=jax.ShapeDtypeStruct((N_USER, DIM), jnp.float32),
    )(p)


def _dinv(h0_blk, h1_blk):
    d = h0_blk[:, 0:1] + h1_blk[:, 0:1] + 1.0
    return lax.rsqrt(d)


def _scale_body(x_ref, h0_ref, h1_ref, wc_ref, y_ref, self_ref):
    xw = jnp.dot(x_ref[...], wc_ref[...], preferred_element_type=jnp.float32)
    di = _dinv(h0_ref[...], h1_ref[...])
    y_ref[...] = xw * di
    self_ref[...] = xw * (di * di)


def _scale(x, h0, h1, Wc):
    blk, grid = 1000, N_NODES // 1000
    return pl.pallas_call(
        _scale_body,
        grid=(grid,),
        in_specs=[
            pl.BlockSpec((blk, DIM), lambda i: (i, 0)),
            pl.BlockSpec((blk, 8), lambda i: (i, 0)),
            pl.BlockSpec((blk, 8), lambda i: (i, 0)),
            pl.BlockSpec((DIM, DIM), lambda i: (0, 0)),
        ],
        out_specs=[
            pl.BlockSpec((blk, DIM), lambda i: (i, 0)),
            pl.BlockSpec((blk, DIM), lambda i: (i, 0)),
        ],
        out_shape=[
            jax.ShapeDtypeStruct((N_NODES, DIM), jnp.float32),
            jax.ShapeDtypeStruct((N_NODES, DIM), jnp.float32),
        ],
    )(x, h0, h1, Wc)


def _mid_body(slo_ref, shi_ref, self_ref, h0_ref, h1_ref, wc_ref, bc_ref,
              h_ref, y_ref, self1_ref):
    di = _dinv(h0_ref[...], h1_ref[...])
    sagg = jnp.concatenate([slo_ref[...], shi_ref[...]], axis=1)
    h = sagg * di + self_ref[...] + bc_ref[...]
    h_ref[...] = h
    hw = jnp.dot(h, wc_ref[...], preferred_element_type=jnp.float32)
    y_ref[...] = hw * di
    self1_ref[...] = hw * (di * di)


def _mid(s_lo, s_hi, self0, h0, h1, Wc, bc):
    blk, grid = 1000, N_NODES // 1000
    return pl.pallas_call(
        _mid_body,
        grid=(grid,),
        in_specs=[
            pl.BlockSpec((blk, HDIM), lambda i: (i, 0)),
            pl.BlockSpec((blk, HDIM), lambda i: (i, 0)),
            pl.BlockSpec((blk, DIM), lambda i: (i, 0)),
            pl.BlockSpec((blk, 8), lambda i: (i, 0)),
            pl.BlockSpec((blk, 8), lambda i: (i, 0)),
            pl.BlockSpec((DIM, DIM), lambda i: (0, 0)),
            pl.BlockSpec((1, DIM), lambda i: (0, 0)),
        ],
        out_specs=[
            pl.BlockSpec((blk, DIM), lambda i: (i, 0)),
            pl.BlockSpec((blk, DIM), lambda i: (i, 0)),
            pl.BlockSpec((blk, DIM), lambda i: (i, 0)),
        ],
        out_shape=[
            jax.ShapeDtypeStruct((N_NODES, DIM), jnp.float32),
            jax.ShapeDtypeStruct((N_NODES, DIM), jnp.float32),
            jax.ShapeDtypeStruct((N_NODES, DIM), jnp.float32),
        ],
    )(s_lo, s_hi, self0, h0, h1, Wc, bc.reshape(1, DIM))


def _final_body(slo_ref, shi_ref, self_ref, h0_ref, h1_ref, x_ref, h_ref,
                bc_ref, o_ref):
    di = _dinv(h0_ref[...], h1_ref[...])
    sagg = jnp.concatenate([slo_ref[...], shi_ref[...]], axis=1)
    h1c = sagg * di + self_ref[...] + bc_ref[...]
    o_ref[...] = x_ref[...] + h_ref[...] + h1c


def _final(s_lo, s_hi, self1, h0, h1, x, h, bc):
    blk, grid = 1000, N_NODES // 1000
    return pl.pallas_call(
        _final_body,
        grid=(grid,),
        in_specs=[
            pl.BlockSpec((blk, HDIM), lambda i: (i, 0)),
            pl.BlockSpec((blk, HDIM), lambda i: (i, 0)),
            pl.BlockSpec((blk, DIM), lambda i: (i, 0)),
            pl.BlockSpec((blk, 8), lambda i: (i, 0)),
            pl.BlockSpec((blk, 8), lambda i: (i, 0)),
            pl.BlockSpec((blk, DIM), lambda i: (i, 0)),
            pl.BlockSpec((blk, DIM), lambda i: (i, 0)),
            pl.BlockSpec((1, DIM), lambda i: (0, 0)),
        ],
        out_specs=pl.BlockSpec((blk, DIM), lambda i: (i, 0)),
        out_shape=jax.ShapeDtypeStruct((N_NODES, DIM), jnp.float32),
    )(s_lo, s_hi, self1, h0, h1, x, h, bc.reshape(1, DIM))


def kernel(edge_index, features, preference, W1, b1, W2, b2, Wc, bc):
    packed = _pack_edges(edge_index[0], edge_index[1])
    packed = jnp.concatenate(
        [packed, jnp.full((PAD_E - N_EDGES,), PAD_PACKED, jnp.int32)])

    hist = _degree_histogram(packed)
    h0 = hist[0, :N_NODES]
    h1 = hist[1, :N_NODES]

    x_items = _mlp_norm(features, W1, b1, W2, b2)
    x_pref = _norm_rows(preference)
    x = jnp.concatenate([x_pref, x_items], axis=0)

    y0, self0 = _scale(x, h0, h1, Wc)
    y0s = jnp.concatenate([y0[:, :HDIM], y0[:, HDIM:]], axis=0)
    s1 = _edge_aggregate(packed, y0s)
    h, y1, self1 = _mid(s1[0, :N_NODES], s1[1, :N_NODES],
                        self0, h0, h1, Wc, bc)
    y1s = jnp.concatenate([y1[:, :HDIM], y1[:, HDIM:]], axis=0)
    s2 = _edge_aggregate(packed, y1s)
    x_hat = _final(s2[0, :N_NODES], s2[1, :N_NODES],
                   self1, h0, h1, x, h, bc)
    return (x_hat, preference)
